# baseline (speedup 1.0000x reference)
import functools

import jax
import jax.numpy as jnp
from jax import lax
from jax.experimental import pallas as pl
from jax.experimental.pallas import tpu as pltpu
from jax.experimental.pallas import tpu_sc as plsc

_ROWS = 160000
_COLS = 256
_NC = 2
_NS = 16
_NW = _NC * _NS
_ROWS_PER = _ROWS // _NW


def kernel(x_j):
    mesh = plsc.VectorSubcoreMesh(core_axis_name="c", subcore_axis_name="s")

    @functools.partial(
        pl.kernel,
        out_type=jax.ShapeDtypeStruct((_ROWS, _COLS), jnp.float32),
        mesh=mesh,
    )
    def sc_copy(x_hbm, out_hbm):
        wid = lax.axis_index("s") * _NC + lax.axis_index("c")
        base = wid * _ROWS_PER
        pltpu.sync_copy(
            x_hbm.at[pl.ds(base, _ROWS_PER), :],
            out_hbm.at[pl.ds(base, _ROWS_PER), :],
        )

    return sc_copy(x_j)


# speedup vs baseline: 49.2347x; 49.2347x over previous
import jax
import jax.numpy as jnp
from jax.experimental import pallas as pl
from jax.experimental.pallas import tpu as pltpu

_ROWS = 160000
_COLS = 256
_CHUNK = 2000
_NCHUNK = _ROWS // _CHUNK
_IN_AHEAD = 4
_OUT_LAG = 4
_NBUF = _IN_AHEAD + _OUT_LAG


def _copy_body(x_ref, o_ref, bufs, in_sems, out_sems):
    def in_copy(i):
        return pltpu.make_async_copy(
            x_ref.at[pl.ds(i * _CHUNK, _CHUNK), :],
            bufs.at[i % _NBUF],
            in_sems.at[i % _NBUF],
        )

    def out_copy(i):
        return pltpu.make_async_copy(
            bufs.at[i % _NBUF],
            o_ref.at[pl.ds(i * _CHUNK, _CHUNK), :],
            out_sems.at[i % _NBUF],
        )

    for i in range(_IN_AHEAD):
        in_copy(i).start()
    for i in range(_NCHUNK):
        in_copy(i).wait()
        out_copy(i).start()
        if i - _OUT_LAG >= 0:
            out_copy(i - _OUT_LAG).wait()
        if i + _IN_AHEAD < _NCHUNK:
            in_copy(i + _IN_AHEAD).start()
    for i in range(max(_NCHUNK - _OUT_LAG, 0), _NCHUNK):
        out_copy(i).wait()


def kernel(x_j):
    return pl.pallas_call(
        _copy_body,
        out_shape=jax.ShapeDtypeStruct(x_j.shape, x_j.dtype),
        in_specs=[pl.BlockSpec(memory_space=pl.ANY)],
        out_specs=pl.BlockSpec(memory_space=pl.ANY),
        scratch_shapes=[
            pltpu.VMEM((_NBUF, _CHUNK, _COLS), jnp.float32),
            pltpu.SemaphoreType.DMA((_NBUF,)),
            pltpu.SemaphoreType.DMA((_NBUF,)),
        ],
    )(x_j)


# speedup vs baseline: 49.2696x; 1.0007x over previous
import jax
import jax.numpy as jnp
from jax.experimental import pallas as pl
from jax.experimental.pallas import tpu as pltpu

_ROWS = 160000
_COLS = 256
_CHUNK = 2000
_NCHUNK = _ROWS // _CHUNK
_IN_AHEAD = 6
_OUT_LAG = 6
_NBUF = _IN_AHEAD + _OUT_LAG


def _copy_body(x_ref, o_ref, bufs, in_sems, out_sems):
    def in_copy(i):
        return pltpu.make_async_copy(
            x_ref.at[pl.ds(i * _CHUNK, _CHUNK), :],
            bufs.at[i % _NBUF],
            in_sems.at[i % _NBUF],
        )

    def out_copy(i):
        return pltpu.make_async_copy(
            bufs.at[i % _NBUF],
            o_ref.at[pl.ds(i * _CHUNK, _CHUNK), :],
            out_sems.at[i % _NBUF],
        )

    for i in range(_IN_AHEAD):
        in_copy(i).start()
    for i in range(_NCHUNK):
        in_copy(i).wait()
        out_copy(i).start()
        if i - _OUT_LAG >= 0:
            out_copy(i - _OUT_LAG).wait()
        if i + _IN_AHEAD < _NCHUNK:
            in_copy(i + _IN_AHEAD).start()
    for i in range(max(_NCHUNK - _OUT_LAG, 0), _NCHUNK):
        out_copy(i).wait()


def kernel(x_j):
    return pl.pallas_call(
        _copy_body,
        out_shape=jax.ShapeDtypeStruct(x_j.shape, x_j.dtype),
        in_specs=[pl.BlockSpec(memory_space=pl.ANY)],
        out_specs=pl.BlockSpec(memory_space=pl.ANY),
        scratch_shapes=[
            pltpu.VMEM((_NBUF, _CHUNK, _COLS), jnp.float32),
            pltpu.SemaphoreType.DMA((_NBUF,)),
            pltpu.SemaphoreType.DMA((_NBUF,)),
        ],
    )(x_j)
